# baseline (speedup 1.0000x reference)
import functools

import jax
import jax.numpy as jnp
from jax import lax
from jax.experimental import pallas as pl
from jax.experimental.pallas import tpu as pltpu
from jax.experimental.pallas import tpu_sc as plsc

_CHUNK = 512
_LANES = 16


def _take(v, idx):
    return lax.gather(
        v, idx[:, None],
        dimension_numbers=lax.GatherDimensionNumbers(
            offset_dims=(), collapsed_slice_dims=(0,), start_index_map=(0,)),
        slice_sizes=(1,),
        mode=lax.GatherScatterMode.PROMISE_IN_BOUNDS)


def _sc_weight_kernel(c0_hbm, c1_hbm, len_hbm, w_hbm, c0_ref, c1_ref, s_ref,
                      w_ref, len_ref, *, B, L):
    nc = 2
    wid = lax.axis_index("s") * nc + lax.axis_index("c")
    lane = lax.iota(jnp.int32, _LANES)
    lane15 = jnp.full((_LANES,), 15, jnp.int32)
    lane0i = jnp.zeros((_LANES,), jnp.int32)
    n_chunks = L // _LANES

    @pl.when(wid < B)
    def _():
        b = wid
        pltpu.sync_copy(c0_hbm.at[b], c0_ref)
        pltpu.sync_copy(c1_hbm.at[b], c1_ref)
        pltpu.sync_copy(len_hbm.at[b], len_ref)
        length = len_ref[...]

        def shifted_in(v, prev_last):
            sh = _take(v, jnp.maximum(lane - 1, 0))
            return jnp.where(lane == 0, prev_last, sh)

        def fwd(i, carry):
            s_carry, p0, p1, gcnt, pos = carry
            base = i * _LANES
            c0v = c0_ref[pl.ds(base, _LANES)]
            c1v = c1_ref[pl.ds(base, _LANES)]
            diff = (c0v != shifted_in(c0v, p0)) | (c1v != shifted_in(c1v, p1))
            valid = pos < length
            ng = ((pos == 0) | diff) & valid
            gcnt = gcnt + jnp.where(ng, jnp.int32(1), jnp.int32(0))
            s = jnp.where(ng, pos, -1)
            for sh in (1, 2, 4, 8):
                s = jnp.maximum(s, _take(s, jnp.maximum(lane - sh, 0)))
            s = jnp.maximum(s, s_carry)
            s_ref[pl.ds(base, _LANES)] = s
            return (_take(s, lane15), _take(c0v, lane15), _take(c1v, lane15),
                    gcnt, pos + _LANES)

        init = (jnp.full((_LANES,), -1, jnp.int32),
                jnp.full((_LANES,), -1, jnp.int32),
                jnp.full((_LANES,), -1, jnp.int32),
                jnp.zeros((_LANES,), jnp.int32),
                lane)
        _, _, _, gcnt, _ = lax.fori_loop(0, n_chunks, fwd, init)
        for sh in (1, 2, 4, 8):
            rot = lane + sh
            rot = jnp.where(rot > 15, rot - _LANES, rot)
            gcnt = gcnt + _take(gcnt, rot)
        g_f = gcnt.astype(jnp.float32)

        def bwd(j, carry):
            nb_carry, pos = carry
            i = n_chunks - 1 - j
            base = i * _LANES
            c0v = c0_ref[pl.ds(base, _LANES)]
            c1v = c1_ref[pl.ds(base, _LANES)]
            pbase = jnp.maximum(base - _LANES, 0)
            p0 = _take(c0_ref[pl.ds(pbase, _LANES)], lane15)
            p1 = _take(c1_ref[pl.ds(pbase, _LANES)], lane15)
            diff = (c0v != shifted_in(c0v, p0)) | (c1v != shifted_in(c1v, p1))
            valid = pos < length
            ng = ((pos == 0) | diff) & valid
            sfx = jnp.where(ng, pos, L)
            for sh in (1, 2, 4, 8):
                sfx = jnp.minimum(sfx, _take(sfx, jnp.minimum(lane + sh, 15)))
            sfx_ex = jnp.where(lane == 15, nb_carry,
                               _take(sfx, jnp.minimum(lane + 1, 15)))
            nb = jnp.minimum(sfx_ex, nb_carry)
            s = s_ref[pl.ds(base, _LANES)]
            run_len = (jnp.minimum(nb, length) - s).astype(jnp.float32)
            denom = g_f * run_len
            safe = valid & (denom > 0.0)
            w = jnp.where(safe, 1.0 / jnp.where(safe, denom, 1.0), 0.0)
            w_ref[pl.ds(base, _LANES)] = w
            return (_take(jnp.minimum(sfx, nb), lane0i), pos - _LANES)

        lax.fori_loop(0, n_chunks, bwd,
                      (jnp.full((_LANES,), L, jnp.int32),
                       (n_chunks - 1) * _LANES + lane))
        pltpu.sync_copy(w_ref, w_hbm.at[b])


def _sc_weights(c0, c1, lengths_b, B, L):
    mesh = plsc.VectorSubcoreMesh(core_axis_name="c", subcore_axis_name="s")
    return pl.kernel(
        functools.partial(_sc_weight_kernel, B=B, L=L),
        out_type=jax.ShapeDtypeStruct((B, L), jnp.float32),
        mesh=mesh,
        scratch_types=[
            pltpu.VMEM((L,), jnp.int32),
            pltpu.VMEM((L,), jnp.int32),
            pltpu.VMEM((L,), jnp.int32),
            pltpu.VMEM((L,), jnp.float32),
            pltpu.VMEM((_LANES,), jnp.int32),
        ],
    )(c0, c1, lengths_b)


def _pool_kernel(len_ref, w_in, feat_hbm, out_ref, b_of, c_of, *,
                 B, N, L, D, chunk):
    def n_chunks(b):
        return (len_ref[b] + chunk - 1) // chunk

    total = n_chunks(0)
    for i in range(1, B):
        total = total + n_chunks(i)

    def build(j, carry):
        b, c = carry
        b_of[j] = b
        c_of[j] = c
        last = (c + 1) == n_chunks(b)
        return (jnp.where(last, b + 1, b), jnp.where(last, 0, c + 1))

    jax.lax.fori_loop(0, total, build, (jnp.int32(0), jnp.int32(0)))

    def inner(idxs, feat_chunk):
        j = idxs[0]

        @pl.when(j == 0)
        def _():
            out_ref[...] = jnp.zeros_like(out_ref)

        b = b_of[j]
        c = c_of[j]
        w_chunk = w_in[pl.ds(b, 1), pl.ds(c * chunk, chunk)]
        out_ref[pl.ds(b, 1), 0] += jnp.dot(w_chunk, feat_chunk[0, 0],
                                           preferred_element_type=jnp.float32)

    pipe = pltpu.emit_pipeline(
        inner,
        grid=(total,),
        in_specs=[pl.BlockSpec((1, 1, chunk, D),
                               lambda j: (b_of[j], N - 1, c_of[j], 0),
                               pipeline_mode=pl.Buffered(buffer_count=4))],
        _explicit_indices=True,
    )
    pipe(feat_hbm)


@jax.jit
def kernel(input_feature, input_lengths, vq_indices):
    B, N, L, D = input_feature.shape
    lengths = input_lengths.astype(jnp.int32)
    vq_t = jnp.transpose(vq_indices.astype(jnp.int32), (0, 2, 1))
    c0 = vq_t[:, 0, :]
    c1 = vq_t[:, 1, :]
    lengths_b = jnp.tile(lengths[:, None], (1, _LANES))

    w = _sc_weights(c0, c1, lengths_b, B, L)

    max_steps = B * (L // _CHUNK)
    grid_spec = pltpu.PrefetchScalarGridSpec(
        num_scalar_prefetch=1,
        grid=(1,),
        in_specs=[
            pl.BlockSpec((B, L), lambda g, lens: (0, 0)),
            pl.BlockSpec(memory_space=pl.ANY),
        ],
        out_specs=pl.BlockSpec((B, 1, D), lambda g, lens: (0, 0, 0)),
        scratch_shapes=[
            pltpu.SMEM((max_steps,), jnp.int32),
            pltpu.SMEM((max_steps,), jnp.int32),
        ],
    )
    out = pl.pallas_call(
        functools.partial(_pool_kernel, B=B, N=N, L=L, D=D, chunk=_CHUNK),
        grid_spec=grid_spec,
        out_shape=jax.ShapeDtypeStruct((B, 1, D), jnp.float32),
    )(lengths, w, input_feature)
    return out[:, 0, :]


# speedup vs baseline: 2.3841x; 2.3841x over previous
import functools

import jax
import jax.numpy as jnp
from jax.experimental import pallas as pl
from jax.experimental.pallas import tpu as pltpu

_CHUNK = 512


def _weights_all(c0, c1, lengths, L):
    B = c0.shape[0]
    idx = jax.lax.broadcasted_iota(jnp.int32, (B, L), 1)
    valid = idx < lengths
    same = (c0 == pltpu.roll(c0, 1, axis=1)) & (c1 == pltpu.roll(c1, 1, axis=1))
    ng = ((idx == 0) | jnp.logical_not(same)) & valid

    s = jnp.where(ng, idx, -1)
    k = 1
    while k < L:
        s = jnp.maximum(s, jnp.where(idx >= k, pltpu.roll(s, k, axis=1), -1))
        k *= 2
    t = jnp.where(ng, idx, L)
    t = jnp.where(idx < L - 1, pltpu.roll(t, L - 1, axis=1), L)
    k = 1
    while k < L:
        t = jnp.minimum(t, jnp.where(idx < L - k, pltpu.roll(t, L - k, axis=1), L))
        k *= 2

    run_len = (jnp.minimum(t, lengths) - s).astype(jnp.float32)
    num_groups = jnp.sum(ng.astype(jnp.float32), axis=1, keepdims=True)
    denom = num_groups * run_len
    safe = valid & (denom > 0.0)
    return jnp.where(safe, 1.0 / jnp.where(safe, denom, 1.0), 0.0)


def _pool_kernel(len_ref, vq_ref, feat_hbm, out_ref, w_ref, b_of, c_of, *,
                 B, N, L, D, chunk):
    def n_chunks(b):
        return (len_ref[b] + chunk - 1) // chunk

    total = n_chunks(0)
    for i in range(1, B):
        total = total + n_chunks(i)

    def build(j, carry):
        b, c = carry
        b_of[j] = b
        c_of[j] = c
        last = (c + 1) == n_chunks(b)
        return (jnp.where(last, b + 1, b), jnp.where(last, 0, c + 1))

    jax.lax.fori_loop(0, total, build, (jnp.int32(0), jnp.int32(0)))

    def inner(idxs, feat_chunk):
        j = idxs[0]

        @pl.when(j == 0)
        def _():
            c0 = vq_ref[:, 0, :]
            c1 = vq_ref[:, 1, :]
            lengths = jnp.concatenate(
                [jnp.full((1, 1), len_ref[i], jnp.int32) for i in range(B)],
                axis=0)
            w_ref[...] = _weights_all(c0, c1, lengths, L)
            out_ref[...] = jnp.zeros_like(out_ref)

        b = b_of[j]
        c = c_of[j]
        w_chunk = w_ref[pl.ds(b, 1), pl.ds(c * chunk, chunk)]
        out_ref[pl.ds(b, 1), 0] += jnp.dot(w_chunk, feat_chunk[0, 0],
                                           preferred_element_type=jnp.float32)

    pipe = pltpu.emit_pipeline(
        inner,
        grid=(total,),
        in_specs=[pl.BlockSpec((1, 1, chunk, D),
                               lambda j: (b_of[j], N - 1, c_of[j], 0),
                               pipeline_mode=pl.Buffered(buffer_count=4, use_lookahead=True))],
        _explicit_indices=True,
    )
    pipe(feat_hbm)


@jax.jit
def kernel(input_feature, input_lengths, vq_indices):
    B, N, L, D = input_feature.shape
    lengths = input_lengths.astype(jnp.int32)
    vq_t = jnp.transpose(vq_indices.astype(jnp.int32), (0, 2, 1))

    max_steps = B * (L // _CHUNK)
    grid_spec = pltpu.PrefetchScalarGridSpec(
        num_scalar_prefetch=1,
        grid=(1,),
        in_specs=[
            pl.BlockSpec((B, 2, L), lambda g, lens: (0, 0, 0)),
            pl.BlockSpec(memory_space=pl.ANY),
        ],
        out_specs=pl.BlockSpec((B, 1, D), lambda g, lens: (0, 0, 0)),
        scratch_shapes=[
            pltpu.VMEM((B, L), jnp.float32),
            pltpu.SMEM((max_steps,), jnp.int32),
            pltpu.SMEM((max_steps,), jnp.int32),
        ],
    )
    out = pl.pallas_call(
        functools.partial(_pool_kernel, B=B, N=N, L=L, D=D, chunk=_CHUNK),
        grid_spec=grid_spec,
        out_shape=jax.ShapeDtypeStruct((B, 1, D), jnp.float32),
    )(lengths, vq_t, input_feature)
    return out[:, 0, :]
